# Initial kernel scaffold; baseline (speedup 1.0000x reference)
import jax
import jax.numpy as jnp
from jax.experimental import pallas as pl


def kernel(embeddings, position_ids):
    raise NotImplementedError("write your pallas kernel here")


# speedup vs baseline: 3.1062x; 3.1062x over previous
import functools

import jax
import jax.numpy as jnp
from jax.experimental import pallas as pl
from jax.experimental.pallas import tpu as pltpu

B, N, D, M = 4, 4096, 1024, 1024
CK = 512
NK = N // CK


def _body(ids_ref, e_ref, o_ref, cnt_ref):
    k = pl.program_id(1)

    @pl.when(k == 0)
    def _():
        o_ref[...] = jnp.zeros_like(o_ref)
        cnt_ref[...] = jnp.zeros_like(cnt_ref)

    ids = ids_ref[0, :, pl.ds(k * CK, CK)]
    mask = (jax.lax.broadcasted_iota(jnp.int32, (M, CK), 0) == ids).astype(
        jnp.float32
    )
    o_ref[0] += jnp.dot(mask, e_ref[0], preferred_element_type=jnp.float32)
    cnt_ref[...] += jnp.broadcast_to(
        jnp.sum(mask, axis=1, keepdims=True), (M, 128)
    )

    @pl.when(k == NK - 1)
    def _():
        o_ref[0] = o_ref[0] / (cnt_ref[:, 0:1] + 1.0)


def kernel(embeddings, position_ids):
    ids3 = position_ids.reshape(B, 1, N)
    out = pl.pallas_call(
        _body,
        grid=(B, NK),
        in_specs=[
            pl.BlockSpec((1, 1, N), lambda b, k: (b, 0, 0)),
            pl.BlockSpec((1, CK, D), lambda b, k: (b, k, 0)),
        ],
        out_specs=pl.BlockSpec((1, M, D), lambda b, k: (b, 0, 0)),
        out_shape=jax.ShapeDtypeStruct((B, M, D), jnp.float32),
        scratch_shapes=[pltpu.VMEM((M, 128), jnp.float32)],
    )(ids3, embeddings)
    return out
